# baseline (speedup 1.0000x reference)
import functools

import jax
import jax.numpy as jnp
from jax import lax
from jax.experimental import pallas as pl
from jax.experimental.pallas import tpu as pltpu, tpu_sc as plsc

N_NODES = 10000
K_NEIGH = 32
KEEP = (14, 15, 16, 17)


def _batcher_comparators(n):
    comps = []
    p = 1
    while p < n:
        k = p
        while k >= 1:
            for j in range(k % p, n - k, 2 * k):
                for i in range(k):
                    if (i + j) // (2 * p) == (i + j + k) // (2 * p):
                        comps.append((i + j, i + j + k))
            k //= 2
        p *= 2
    return comps


def _pruned_network(n, keep):
    live = set(keep)
    out = []
    for a, b in reversed(_batcher_comparators(n)):
        need_min, need_max = a in live, b in live
        if need_min or need_max:
            live.add(a)
            live.add(b)
            out.append((a, b, need_min, need_max))
    out.reverse()
    return out


_NETWORK = _pruned_network(K_NEIGH, KEEP)


def _mid4_sum(vals):
    w = list(vals)
    for a, b, need_min, need_max in _NETWORK:
        va, vb = w[a], w[b]
        if need_min:
            w[a] = jnp.minimum(va, vb)
        if need_max:
            w[b] = jnp.maximum(va, vb)
    return (w[14] + w[15]) + (w[16] + w[17])


def _make_sc_agg(n_pad, d, relu, paired):
    info = plsc.get_sparse_core_info()
    npw0, npw1 = 320, 320
    assert npw0 * info.num_subcores + npw1 * info.num_subcores == n_pad
    ch = 32
    chb = ch * K_NEIGH // 128
    nch0, nch1 = npw0 // ch, npw1 // ch
    mesh = plsc.VectorSubcoreMesh(core_axis_name="c", subcore_axis_name="s")

    idx_n0 = npw0 * K_NEIGH

    scratch = [
        pltpu.VMEM((idx_n0,), jnp.int32),
        pltpu.VMEM((ch * K_NEIGH, d), jnp.bfloat16),
        pltpu.VMEM((ch * K_NEIGH, d), jnp.bfloat16),
        pltpu.VMEM((npw0, d), jnp.bfloat16),
        pltpu.SemaphoreType.DMA,
        pltpu.SemaphoreType.DMA,
    ]
    if paired:
        scratch.append(pltpu.VMEM((idx_n0,), jnp.int32))

    @functools.partial(
        pl.kernel,
        mesh=mesh,
        compiler_params=pltpu.CompilerParams(use_tc_tiling_on_sc=False,
                                             needs_layout_passes=False),
        out_type=jax.ShapeDtypeStruct((n_pad, d), jnp.bfloat16),
        scratch_types=scratch,
    )
    def agg(adj_hbm, table_hbm, out_hbm, idx_v, rows0, rows1, out_v, s0, s1,
            *maybe_il):
        cid = lax.axis_index("c")
        sid = lax.axis_index("s")
        on0 = cid == 0
        wbase = jnp.where(on0, sid * npw0,
                          info.num_subcores * npw0 + sid * npw1)
        nch = jnp.where(on0, nch0, nch1)
        bufs = ((rows0, s0), (rows1, s1))

        @pl.when(on0)
        def _():
            pltpu.sync_copy(
                adj_hbm.at[pl.ds(pl.multiple_of(wbase * K_NEIGH, 8), idx_n0)],
                idx_v.at[pl.ds(0, idx_n0)])

        @pl.when(jnp.logical_not(on0))
        def _():
            pltpu.sync_copy(
                adj_hbm.at[pl.ds(pl.multiple_of(wbase * K_NEIGH, 8),
                                 npw1 * K_NEIGH)],
                idx_v.at[pl.ds(0, npw1 * K_NEIGH)])

        if paired:
            idx_use = maybe_il[0]
            iota = lax.iota(jnp.int32, 16)
            pv = (iota & 1) * 32 + (iota >> 1)

            def il_body(v, carry):
                p0 = v * 16
                rem = p0 & 63
                base = p0 - rem + (rem >> 1)
                idx_use[pl.ds(p0, 16)] = plsc.load_gather(idx_v, [pv + base])
                return carry

            lax.fori_loop(0, nch * (ch * K_NEIGH // 16), il_body, 0)
        else:
            idx_use = idx_v

        def gather_descs(ci, b):
            rows_v, sem = bufs[b]
            return [pltpu.make_async_copy(
                table_hbm.at[idx_use.at[pl.ds(ci * ch * K_NEIGH + j * 128,
                                              128)]],
                rows_v.at[pl.ds(j * 128, 128)], sem) for j in range(chb)]

        def issue(ci, b):
            for dsc in gather_descs(ci, b):
                dsc.start()

        def drain(ci, b):
            for dsc in gather_descs(ci, b):
                dsc.wait()

        def compute(ci, b):
            rows_v, _ = bufs[b]
            if paired:
                @plsc.parallel_loop(0, ch // 2, unroll=2)
                def _(c):
                    vals = [rows_v[pl.ds(c * 2 * K_NEIGH + 2 * k, 2), :]
                            for k in range(K_NEIGH)]
                    s = _mid4_sum(vals) * 0.25
                    if relu:
                        s = jnp.maximum(s, 0.0)
                    out_v[pl.ds(ci * ch + 2 * c, 2), :] = s
            else:
                @plsc.parallel_loop(0, ch, unroll=2)
                def _(c):
                    for g in range(d // 32):
                        vals = [rows_v[c * K_NEIGH + k, pl.ds(g * 32, 32)]
                                for k in range(K_NEIGH)]
                        s = _mid4_sum(vals) * 0.25
                        if relu:
                            s = jnp.maximum(s, 0.0)
                        out_v[ci * ch + c, pl.ds(g * 32, 32)] = s

        issue(0, 0)

        def pair_body(i2, carry):
            ci0 = 2 * i2
            drain(ci0, 0)
            issue(ci0 + 1, 1)
            compute(ci0, 0)
            drain(ci0 + 1, 1)

            @pl.when(i2 < nch // 2 - 1)
            def _():
                issue(ci0 + 2, 0)

            compute(ci0 + 1, 1)
            return carry

        lax.fori_loop(0, nch // 2, pair_body, 0)

        @pl.when(on0)
        def _():
            pltpu.sync_copy(out_v.at[pl.ds(0, npw0)],
                            out_hbm.at[pl.ds(wbase, npw0)])

        @pl.when(jnp.logical_not(on0))
        def _():
            pltpu.sync_copy(out_v.at[pl.ds(0, npw1)],
                            out_hbm.at[pl.ds(wbase, npw1)])

    return agg


def _mm(a, w):
    m, kd = a.shape
    nd = w.shape[1]
    bm = 1024 if m % 1024 == 0 else 1000

    def body(a_ref, w_ref, o_ref):
        o_ref[...] = jnp.dot(
            a_ref[...].astype(jnp.float32), w_ref[...],
            preferred_element_type=jnp.float32).astype(jnp.bfloat16)

    return pl.pallas_call(
        body,
        grid=(m // bm,),
        in_specs=[pl.BlockSpec((bm, kd), lambda i: (i, 0)),
                  pl.BlockSpec((kd, nd), lambda i: (0, 0))],
        out_specs=pl.BlockSpec((bm, nd), lambda i: (i, 0)),
        out_shape=jax.ShapeDtypeStruct((m, nd), jnp.bfloat16),
    )(a, w)


def kernel(x, adj, W1, W2):
    n_pad = 10240
    pad_idx = (jnp.arange((n_pad - N_NODES) * K_NEIGH, dtype=jnp.int32)
               * 131 % N_NODES).reshape(n_pad - N_NODES, K_NEIGH)
    adj_flat = jnp.concatenate([adj, pad_idx], axis=0).reshape(-1)

    h1 = _mm(x, W1)
    a1 = _make_sc_agg(n_pad, 16, relu=True, paired=True)(adj_flat, h1)
    h2 = _mm(a1, W2)
    out = _make_sc_agg(n_pad, 64, relu=False, paired=False)(adj_flat, h2)
    return out[:N_NODES].astype(jnp.float32)


# speedup vs baseline: 1.0183x; 1.0183x over previous
import functools

import jax
import jax.numpy as jnp
from jax import lax
from jax.experimental import pallas as pl
from jax.experimental.pallas import tpu as pltpu, tpu_sc as plsc

N_NODES = 10000
K_NEIGH = 32
KEEP = (14, 15, 16, 17)


def _batcher_comparators(n):
    comps = []
    p = 1
    while p < n:
        k = p
        while k >= 1:
            for j in range(k % p, n - k, 2 * k):
                for i in range(k):
                    if (i + j) // (2 * p) == (i + j + k) // (2 * p):
                        comps.append((i + j, i + j + k))
            k //= 2
        p *= 2
    return comps


def _pruned_network(n, keep):
    live = set(keep)
    out = []
    for a, b in reversed(_batcher_comparators(n)):
        need_min, need_max = a in live, b in live
        if need_min or need_max:
            live.add(a)
            live.add(b)
            out.append((a, b, need_min, need_max))
    out.reverse()
    return out


_NETWORK = _pruned_network(K_NEIGH, KEEP)


def _mid4_sum(vals):
    w = list(vals)
    for a, b, need_min, need_max in _NETWORK:
        va, vb = w[a], w[b]
        if need_min:
            w[a] = jnp.minimum(va, vb)
        if need_max:
            w[b] = jnp.maximum(va, vb)
    return (w[14] + w[15]) + (w[16] + w[17])


def _make_sc_agg(n_pad, d, relu, paired):
    info = plsc.get_sparse_core_info()
    npw0, npw1 = 320, 320
    assert npw0 * info.num_subcores + npw1 * info.num_subcores == n_pad
    ch = 40
    chb = ch * K_NEIGH // 128
    nch0, nch1 = npw0 // ch, npw1 // ch
    mesh = plsc.VectorSubcoreMesh(core_axis_name="c", subcore_axis_name="s")

    idx_n0 = npw0 * K_NEIGH

    scratch = [
        pltpu.VMEM((idx_n0,), jnp.int32),
        pltpu.VMEM((ch * K_NEIGH, d), jnp.bfloat16),
        pltpu.VMEM((ch * K_NEIGH, d), jnp.bfloat16),
        pltpu.VMEM((npw0, d), jnp.bfloat16),
        pltpu.SemaphoreType.DMA,
        pltpu.SemaphoreType.DMA,
    ]
    if paired:
        scratch.append(pltpu.VMEM((idx_n0,), jnp.int32))

    @functools.partial(
        pl.kernel,
        mesh=mesh,
        compiler_params=pltpu.CompilerParams(use_tc_tiling_on_sc=False,
                                             needs_layout_passes=False),
        out_type=jax.ShapeDtypeStruct((n_pad, d), jnp.bfloat16),
        scratch_types=scratch,
    )
    def agg(adj_hbm, table_hbm, out_hbm, idx_v, rows0, rows1, out_v, s0, s1,
            *maybe_il):
        cid = lax.axis_index("c")
        sid = lax.axis_index("s")
        on0 = cid == 0
        wbase = jnp.where(on0, sid * npw0,
                          info.num_subcores * npw0 + sid * npw1)
        nch = jnp.where(on0, nch0, nch1)
        bufs = ((rows0, s0), (rows1, s1))

        @pl.when(on0)
        def _():
            pltpu.sync_copy(
                adj_hbm.at[pl.ds(pl.multiple_of(wbase * K_NEIGH, 8), idx_n0)],
                idx_v.at[pl.ds(0, idx_n0)])

        @pl.when(jnp.logical_not(on0))
        def _():
            pltpu.sync_copy(
                adj_hbm.at[pl.ds(pl.multiple_of(wbase * K_NEIGH, 8),
                                 npw1 * K_NEIGH)],
                idx_v.at[pl.ds(0, npw1 * K_NEIGH)])

        if paired:
            idx_use = maybe_il[0]
            iota = lax.iota(jnp.int32, 16)
            pv = (iota & 1) * 32 + (iota >> 1)

            def il_body(v, carry):
                p0 = v * 16
                rem = p0 & 63
                base = p0 - rem + (rem >> 1)
                idx_use[pl.ds(p0, 16)] = plsc.load_gather(idx_v, [pv + base])
                return carry

            lax.fori_loop(0, nch * (ch * K_NEIGH // 16), il_body, 0)
        else:
            idx_use = idx_v

        def gather_descs(ci, b):
            rows_v, sem = bufs[b]
            return [pltpu.make_async_copy(
                table_hbm.at[idx_use.at[pl.ds(ci * ch * K_NEIGH + j * 128,
                                              128)]],
                rows_v.at[pl.ds(j * 128, 128)], sem) for j in range(chb)]

        def issue(ci, b):
            for dsc in gather_descs(ci, b):
                dsc.start()

        def drain(ci, b):
            for dsc in gather_descs(ci, b):
                dsc.wait()

        def compute(ci, b):
            rows_v, _ = bufs[b]
            if paired:
                @plsc.parallel_loop(0, ch // 2)
                def _(c):
                    vals = [rows_v[pl.ds(c * 2 * K_NEIGH + 2 * k, 2), :]
                            for k in range(K_NEIGH)]
                    s = _mid4_sum(vals) * 0.25
                    if relu:
                        s = jnp.maximum(s, 0.0)
                    out_v[pl.ds(ci * ch + 2 * c, 2), :] = s
            else:
                @plsc.parallel_loop(0, ch)
                def _(c):
                    for g in range(d // 32):
                        vals = [rows_v[c * K_NEIGH + k, pl.ds(g * 32, 32)]
                                for k in range(K_NEIGH)]
                        s = _mid4_sum(vals) * 0.25
                        if relu:
                            s = jnp.maximum(s, 0.0)
                        out_v[ci * ch + c, pl.ds(g * 32, 32)] = s

        issue(0, 0)

        def pair_body(i2, carry):
            ci0 = 2 * i2
            drain(ci0, 0)
            issue(ci0 + 1, 1)
            compute(ci0, 0)
            drain(ci0 + 1, 1)

            @pl.when(i2 < nch // 2 - 1)
            def _():
                issue(ci0 + 2, 0)

            compute(ci0 + 1, 1)
            return carry

        lax.fori_loop(0, nch // 2, pair_body, 0)

        @pl.when(on0)
        def _():
            pltpu.sync_copy(out_v.at[pl.ds(0, npw0)],
                            out_hbm.at[pl.ds(wbase, npw0)])

        @pl.when(jnp.logical_not(on0))
        def _():
            pltpu.sync_copy(out_v.at[pl.ds(0, npw1)],
                            out_hbm.at[pl.ds(wbase, npw1)])

    return agg


def _mm(a, w):
    m, kd = a.shape
    nd = w.shape[1]
    bm = 1024 if m % 1024 == 0 else 1000

    def body(a_ref, w_ref, o_ref):
        o_ref[...] = jnp.dot(
            a_ref[...].astype(jnp.float32), w_ref[...],
            preferred_element_type=jnp.float32).astype(jnp.bfloat16)

    return pl.pallas_call(
        body,
        grid=(m // bm,),
        in_specs=[pl.BlockSpec((bm, kd), lambda i: (i, 0)),
                  pl.BlockSpec((kd, nd), lambda i: (0, 0))],
        out_specs=pl.BlockSpec((bm, nd), lambda i: (i, 0)),
        out_shape=jax.ShapeDtypeStruct((m, nd), jnp.bfloat16),
    )(a, w)


def kernel(x, adj, W1, W2):
    n_pad = 10240
    pad_idx = (jnp.arange((n_pad - N_NODES) * K_NEIGH, dtype=jnp.int32)
               * 131 % N_NODES).reshape(n_pad - N_NODES, K_NEIGH)
    adj_flat = jnp.concatenate([adj, pad_idx], axis=0).reshape(-1)

    h1 = _mm(x, W1)
    a1 = _make_sc_agg(n_pad, 16, relu=True, paired=True)(adj_flat, h1)
    h2 = _mm(a1, W2)
    out = _make_sc_agg(n_pad, 64, relu=False, paired=False)(adj_flat, h2)
    return out[:N_NODES].astype(jnp.float32)


# speedup vs baseline: 1.0765x; 1.0572x over previous
import functools

import jax
import jax.numpy as jnp
from jax import lax
from jax.experimental import pallas as pl
from jax.experimental.pallas import tpu as pltpu, tpu_sc as plsc

N_NODES = 10000
K_NEIGH = 32
KEEP = (14, 15, 16, 17)


def _batcher_comparators(n):
    comps = []
    p = 1
    while p < n:
        k = p
        while k >= 1:
            for j in range(k % p, n - k, 2 * k):
                for i in range(k):
                    if (i + j) // (2 * p) == (i + j + k) // (2 * p):
                        comps.append((i + j, i + j + k))
            k //= 2
        p *= 2
    return comps


def _pruned_network(n, keep):
    live = set(keep)
    out = []
    for a, b in reversed(_batcher_comparators(n)):
        need_min, need_max = a in live, b in live
        if need_min or need_max:
            live.add(a)
            live.add(b)
            out.append((a, b, need_min, need_max))
    out.reverse()
    return out


_NETWORK = _pruned_network(K_NEIGH, KEEP)


def _mid4_sum(vals):
    w = list(vals)
    for a, b, need_min, need_max in _NETWORK:
        va, vb = w[a], w[b]
        if need_min:
            w[a] = jnp.minimum(va, vb)
        if need_max:
            w[b] = jnp.maximum(va, vb)
    return (w[14] + w[15]) + (w[16] + w[17])


def _make_sc_agg(n_pad, d, relu, paired):
    info = plsc.get_sparse_core_info()
    npw0, npw1 = 320, 320
    assert npw0 * info.num_subcores + npw1 * info.num_subcores == n_pad
    ch = 40
    chb = ch * K_NEIGH // 128
    nch0, nch1 = npw0 // ch, npw1 // ch
    mesh = plsc.VectorSubcoreMesh(core_axis_name="c", subcore_axis_name="s")

    idx_n0 = npw0 * K_NEIGH

    scratch = [
        pltpu.VMEM((idx_n0,), jnp.int32),
        pltpu.VMEM((ch * K_NEIGH, d), jnp.bfloat16),
        pltpu.VMEM((ch * K_NEIGH, d), jnp.bfloat16),
        pltpu.VMEM((npw0, d), jnp.bfloat16),
        pltpu.SemaphoreType.DMA,
        pltpu.SemaphoreType.DMA,
    ]
    if paired:
        scratch.append(pltpu.VMEM((idx_n0,), jnp.int32))

    @functools.partial(
        pl.kernel,
        mesh=mesh,
        compiler_params=pltpu.CompilerParams(use_tc_tiling_on_sc=False,
                                             needs_layout_passes=False),
        out_type=jax.ShapeDtypeStruct((n_pad, d), jnp.bfloat16),
        scratch_types=scratch,
    )
    def agg(adj_hbm, table_hbm, out_hbm, idx_v, rows0, rows1, out_v, s0, s1,
            *maybe_il):
        cid = lax.axis_index("c")
        sid = lax.axis_index("s")
        on0 = cid == 0
        wbase = jnp.where(on0, sid * npw0,
                          info.num_subcores * npw0 + sid * npw1)
        nch = jnp.where(on0, nch0, nch1)
        bufs = ((rows0, s0), (rows1, s1))

        @pl.when(on0)
        def _():
            pltpu.sync_copy(
                adj_hbm.at[pl.ds(pl.multiple_of(wbase * K_NEIGH, 8), idx_n0)],
                idx_v.at[pl.ds(0, idx_n0)])

        @pl.when(jnp.logical_not(on0))
        def _():
            pltpu.sync_copy(
                adj_hbm.at[pl.ds(pl.multiple_of(wbase * K_NEIGH, 8),
                                 npw1 * K_NEIGH)],
                idx_v.at[pl.ds(0, npw1 * K_NEIGH)])

        if paired:
            idx_use = maybe_il[0]
            iota = lax.iota(jnp.int32, 16)
            pv = (iota & 1) * 32 + (iota >> 1)

            def il_body(v, carry):
                p0 = v * 16
                rem = p0 & 63
                base = p0 - rem + (rem >> 1)
                idx_use[pl.ds(p0, 16)] = plsc.load_gather(idx_v, [pv + base])
                return carry

            lax.fori_loop(0, nch * (ch * K_NEIGH // 16), il_body, 0)
        else:
            idx_use = idx_v

        def gather_descs(ci, b):
            rows_v, sem = bufs[b]
            return [pltpu.make_async_copy(
                table_hbm.at[idx_use.at[pl.ds(ci * ch * K_NEIGH + j * 128,
                                              128)]],
                rows_v.at[pl.ds(j * 128, 128)], sem) for j in range(chb)]

        def issue(ci, b):
            for dsc in gather_descs(ci, b):
                dsc.start()

        def drain(ci, b):
            for dsc in gather_descs(ci, b):
                dsc.wait()

        def compute(ci, b):
            rows_v, _ = bufs[b]
            if paired:
                @plsc.parallel_loop(0, ch // 2)
                def _(c):
                    vals = [rows_v[pl.ds(c * 2 * K_NEIGH + 2 * k, 2), :]
                            for k in range(K_NEIGH)]
                    s = _mid4_sum(vals) * 0.25
                    if relu:
                        s = jnp.maximum(s, 0.0)
                    out_v[pl.ds(ci * ch + 2 * c, 2), :] = s
            else:
                @plsc.parallel_loop(0, ch)
                def _(c):
                    for g in range(d // 32):
                        vals = [rows_v[c * K_NEIGH + k, pl.ds(g * 32, 32)]
                                for k in range(K_NEIGH)]
                        s = _mid4_sum(vals) * 0.25
                        if relu:
                            s = jnp.maximum(s, 0.0)
                        out_v[ci * ch + c, pl.ds(g * 32, 32)] = s

        issue(0, 0)

        def pair_body(i2, carry):
            ci0 = 2 * i2
            drain(ci0, 0)
            issue(ci0 + 1, 1)
            compute(ci0, 0)
            drain(ci0 + 1, 1)

            @pl.when(i2 < nch // 2 - 1)
            def _():
                issue(ci0 + 2, 0)

            compute(ci0 + 1, 1)
            return carry

        lax.fori_loop(0, nch // 2, pair_body, 0)

        @pl.when(on0)
        def _():
            pltpu.sync_copy(out_v.at[pl.ds(0, npw0)],
                            out_hbm.at[pl.ds(wbase, npw0)])

        @pl.when(jnp.logical_not(on0))
        def _():
            pltpu.sync_copy(out_v.at[pl.ds(0, npw1)],
                            out_hbm.at[pl.ds(wbase, npw1)])

    return agg


def _mm(a, w):
    m, kd = a.shape
    nd = w.shape[1]
    bm = m

    def body(a_ref, w_ref, o_ref):
        o_ref[...] = jnp.dot(
            a_ref[...].astype(jnp.float32), w_ref[...],
            preferred_element_type=jnp.float32).astype(jnp.bfloat16)

    return pl.pallas_call(
        body,
        grid=(m // bm,),
        in_specs=[pl.BlockSpec((bm, kd), lambda i: (i, 0)),
                  pl.BlockSpec((kd, nd), lambda i: (0, 0))],
        out_specs=pl.BlockSpec((bm, nd), lambda i: (i, 0)),
        out_shape=jax.ShapeDtypeStruct((m, nd), jnp.bfloat16),
    )(a, w)


def kernel(x, adj, W1, W2):
    n_pad = 10240
    pad_idx = (jnp.arange((n_pad - N_NODES) * K_NEIGH, dtype=jnp.int32)
               * 131 % N_NODES).reshape(n_pad - N_NODES, K_NEIGH)
    adj_flat = jnp.concatenate([adj, pad_idx], axis=0).reshape(-1)

    h1 = _mm(x, W1)
    a1 = _make_sc_agg(n_pad, 16, relu=True, paired=True)(adj_flat, h1)
    h2 = _mm(a1, W2)
    out = _make_sc_agg(n_pad, 64, relu=False, paired=False)(adj_flat, h2)
    return out[:N_NODES].astype(jnp.float32)
